# Initial kernel scaffold; baseline (speedup 1.0000x reference)
import jax
import jax.numpy as jnp
from jax.experimental import pallas as pl


def kernel(x, edge_index, W1, att_src1, att_dst1, b1, W2, att_src2, att_dst2, b2):
    raise NotImplementedError("write your pallas kernel here")


# speedup vs baseline: 13.3247x; 13.3247x over previous
import dataclasses
import functools

import jax
import jax.numpy as jnp
from jax import lax
from jax.experimental import pallas as pl
from jax.experimental.pallas import tpu as pltpu
from jax.experimental.pallas import tpu_sc as plsc

N_IN = 64
HEADS = 8
HID = 64
F = HEADS * HID
FT = 640

BN = 128
NB = 416
NP = NB * BN
NWORK = 32
BPW = NB // NWORK
CE = 64
CE2 = 128
BA = 256

_f32 = jnp.float32
_i32 = jnp.int32


def _ka_body(x_ref, w_ref, asf_ref, adf_ref, ht_ref, ad_ref,
             ms_ref, md_ref):
    h = jnp.dot(x_ref[...], w_ref[...], preferred_element_type=_f32)
    ht_ref[:, :F] = h
    ts = h * asf_ref[...]
    td = h * adf_ref[...]
    zs = jnp.zeros((BA, 8), _f32)
    asb = jnp.concatenate(
        [jnp.sum(ts[:, i * HID:(i + 1) * HID], axis=1, keepdims=True)
         for i in range(HEADS)] + [zs], axis=1)
    adb = jnp.concatenate(
        [jnp.sum(td[:, i * HID:(i + 1) * HID], axis=1, keepdims=True)
         for i in range(HEADS)] + [zs], axis=1)
    ht_ref[:, F:F + 16] = asb
    ht_ref[:, F + 16:] = jnp.zeros((BA, FT - F - 16), _f32)
    ad_ref[...] = adb
    msb = jnp.max(asb, axis=0, keepdims=True)
    mdb = jnp.max(adb, axis=0, keepdims=True)
    i = pl.program_id(0)

    @pl.when(i == 0)
    def _():
        ms_ref[...] = msb
        md_ref[...] = mdb

    @pl.when(i != 0)
    def _():
        ms_ref[...] = jnp.maximum(ms_ref[...], msb)
        md_ref[...] = jnp.maximum(md_ref[...], mdb)


def _ka(x_p, w1, asf, adf):
    return pl.pallas_call(
        _ka_body,
        grid=(NP // BA,),
        in_specs=[
            pl.BlockSpec((BA, N_IN), lambda i: (i, 0)),
            pl.BlockSpec((N_IN, F), lambda i: (0, 0)),
            pl.BlockSpec((1, F), lambda i: (0, 0)),
            pl.BlockSpec((1, F), lambda i: (0, 0)),
        ],
        out_specs=[
            pl.BlockSpec((BA, FT), lambda i: (i, 0)),
            pl.BlockSpec((BA, 16), lambda i: (i, 0)),
            pl.BlockSpec((1, 16), lambda i: (0, 0)),
            pl.BlockSpec((1, 16), lambda i: (0, 0)),
        ],
        out_shape=[
            jax.ShapeDtypeStruct((NP, FT), _f32),
            jax.ShapeDtypeStruct((NP, 16), _f32),
            jax.ShapeDtypeStruct((1, 16), _f32),
            jax.ShapeDtypeStruct((1, 16), _f32),
        ],
    )(x_p, w1, asf, adf)


def _kd_body(acc_ref, den_ref, b1_ref, w2_ref, att_ref, h2_ref, m2_ref):
    den = den_ref[...]
    denb = jnp.concatenate(
        [jnp.broadcast_to(den[:, i:i + 1], (BA, HID)) for i in range(HEADS)],
        axis=1)
    y = acc_ref[...] / (denb + 1e-16) + b1_ref[...]
    y = jnp.where(y > 0, y, jnp.exp(jnp.minimum(y, 0.0)) - 1.0)
    h2 = jnp.sum(y * w2_ref[...], axis=1, keepdims=True)
    h2_ref[...] = h2
    s = att_ref[0, 0] * h2
    d = att_ref[0, 1] * h2
    m2b = jnp.concatenate(
        [jnp.max(s, axis=0, keepdims=True),
         jnp.max(d, axis=0, keepdims=True),
         jnp.zeros((1, 14), _f32)], axis=1)
    i = pl.program_id(0)

    @pl.when(i == 0)
    def _():
        m2_ref[...] = m2b

    @pl.when(i != 0)
    def _():
        m2_ref[...] = jnp.maximum(m2_ref[...], m2b)


def _kd(acc1, den1, b1r, w2r, att2):
    return pl.pallas_call(
        _kd_body,
        grid=(NP // BA,),
        in_specs=[
            pl.BlockSpec((BA, F), lambda i: (i, 0)),
            pl.BlockSpec((BA, 16), lambda i: (i, 0)),
            pl.BlockSpec((1, F), lambda i: (0, 0)),
            pl.BlockSpec((1, F), lambda i: (0, 0)),
            pl.BlockSpec((1, 16), lambda i: (0, 0)),
        ],
        out_specs=[
            pl.BlockSpec((BA, 1), lambda i: (i, 0)),
            pl.BlockSpec((1, 16), lambda i: (0, 0)),
        ],
        out_shape=[
            jax.ShapeDtypeStruct((NP, 1), _f32),
            jax.ShapeDtypeStruct((1, 16), _f32),
        ],
    )(acc1, den1, b1r, w2r, att2)


def _kf_body(acc_ref, den_ref, b2_ref, o_ref):
    o_ref[...] = (acc_ref[:, :1] / (den_ref[:, :1] + 1e-16)
                  + b2_ref[0, 0])


def _kf(acc2, den2, b2p):
    return pl.pallas_call(
        _kf_body,
        grid=(NP // BA,),
        in_specs=[
            pl.BlockSpec((BA, 16), lambda i: (i, 0)),
            pl.BlockSpec((BA, 16), lambda i: (i, 0)),
            pl.BlockSpec((1, 16), lambda i: (0, 0)),
        ],
        out_specs=pl.BlockSpec((BA, 1), lambda i: (i, 0)),
        out_shape=jax.ShapeDtypeStruct((NP, 1), _f32),
    )(acc2, den2, b2p)


_mesh = plsc.VectorSubcoreMesh(core_axis_name="c", subcore_axis_name="s")

_sc_params = pltpu.CompilerParams()
if "needs_layout_passes" in pltpu.CompilerParams.__dataclass_fields__:
    _sc_params = dataclasses.replace(_sc_params, needs_layout_passes=False)


@functools.partial(
    pl.kernel,
    out_type=[
        jax.ShapeDtypeStruct((NP, F), _f32),
        jax.ShapeDtypeStruct((NP * 16,), _f32),
    ],
    mesh=_mesh,
    scratch_types=[
        pltpu.VMEM((BN + 1, F), _f32),
        pltpu.VMEM(((BN + 1) * 16,), _f32),
        pltpu.VMEM((CE, FT), _f32),
        pltpu.VMEM((CE,), _i32),
        pltpu.VMEM((CE,), _i32),
        pltpu.VMEM(((BN + 1) * 16,), _f32),
        pltpu.VMEM((16,), _i32),
        pltpu.VMEM((16,), _f32),
        pltpu.SemaphoreType.DMA,
        pltpu.SemaphoreType.DMA,
        pltpu.SemaphoreType.DMA,
    ],
)
def _kb(ht, a1d, srcs, dsts, bpm, scalh, acc1, den1,
        acc_v, den_v, rows_v, sidx_v, didx_v, adb_v,
        bp_v, scal_v, s1, s2, s3):
    w = lax.axis_index("c") * 16 + lax.axis_index("s")
    pltpu.sync_copy(scalh, scal_v)
    z16 = jnp.zeros((16,), _f32)
    cvec = scal_v[...]

    def zero_acc():
        @pl.loop(0, BN + 1)
        def _(d):
            den_v[pl.ds(d * 16, 16)] = z16
            for i in range(F // 16):
                acc_v[d, pl.ds(i * 16, 16)] = z16

    zero_acc()

    def block(t, carry0):
        b = w + NWORK * t
        base = b * BN
        pltpu.sync_copy(bpm.at[b], bp_v)
        pltpu.sync_copy(a1d.at[pl.ds(base * 16, BN * 16)],
                        adb_v.at[pl.ds(0, BN * 16)])
        adb_v[pl.ds(BN * 16, 16)] = z16
        bpl = bp_v[...]
        s_b = bpl[0]
        s_e = bpl[1]
        a0 = s_b & (-8)
        nch = (s_e - a0 + CE - 1) >> 6

        def chunk(ci, carry):
            lo = pl.multiple_of(a0 + ci * CE, 8)
            c1 = pltpu.async_copy(srcs.at[pl.ds(lo, CE)], sidx_v, s1)
            c2 = pltpu.async_copy(dsts.at[pl.ds(lo, CE)], didx_v, s2)
            c1.wait()
            c3 = pltpu.async_copy(ht.at[sidx_v], rows_v, s3)
            c2.wait()
            c3.wait()

            @pl.loop(0, CE // 16)
            def _(g):
                g16 = g * 16
                dvec = didx_v[pl.ds(g16, 16)]
                dl = dvec - base
                jv = jnp.where((dl >= 0) & (dl < BN), dl, BN)
                for l in range(16):
                    j = g16 + l
                    dloc = jv[l]
                    sv = (rows_v[j, pl.ds(F, 16)]
                          + adb_v[pl.ds(dloc * 16, 16)])
                    al = jnp.maximum(sv, 0.2 * sv)
                    exrow = jnp.exp(al - cvec)
                    plsc.addupdate(den_v.at[pl.ds(dloc * 16, 16)], exrow)
                    for h in range(HEADS):
                        exv = jnp.full((16,), exrow[h], _f32)
                        for c in range(HID // 16):
                            off = h * HID + c * 16
                            plsc.addupdate(
                                acc_v.at[dloc, pl.ds(off, 16)],
                                exv * rows_v[j, pl.ds(off, 16)])
            return carry

        lax.fori_loop(0, nch, chunk, 0)
        pltpu.sync_copy(acc_v.at[pl.ds(0, BN)], acc1.at[pl.ds(base, BN)])
        pltpu.sync_copy(den_v.at[pl.ds(0, BN * 16)],
                        den1.at[pl.ds(base * 16, BN * 16)])
        zero_acc()
        return carry0

    lax.fori_loop(0, BPW, block, 0)


@functools.partial(
    pl.kernel,
    out_type=[
        jax.ShapeDtypeStruct((NP * 16,), _f32),
        jax.ShapeDtypeStruct((NP * 16,), _f32),
    ],
    mesh=_mesh,
    scratch_types=[
        pltpu.VMEM((NP,), _f32),
        pltpu.VMEM(((BN + 1) * 16,), _f32),
        pltpu.VMEM(((BN + 1) * 16,), _f32),
        pltpu.VMEM((CE2,), _i32),
        pltpu.VMEM((CE2,), _i32),
        pltpu.VMEM((16,), _i32),
        pltpu.VMEM((16,), _f32),
        pltpu.SemaphoreType.DMA,
        pltpu.SemaphoreType.DMA,
    ],
    compiler_params=_sc_params,
)
def _ke(h2h, srcs, dsts, bpm, scalh, acc2h, den2h,
        h2l_v, acc2_v, den2_v, sidx_v, didx_v,
        bp_v, scal_v, s1, s2):
    w = lax.axis_index("c") * 16 + lax.axis_index("s")
    pltpu.sync_copy(scalh, scal_v)
    pltpu.sync_copy(h2h, h2l_v)
    z16 = jnp.zeros((16,), _f32)
    scl = scal_v[...]
    c2v = jnp.full((16,), scl[0], _f32)
    asv = jnp.full((16,), scl[1], _f32)
    adv = jnp.full((16,), scl[2], _f32)

    def block(t, carry0):
        b = w + NWORK * t
        base = b * BN
        pltpu.sync_copy(bpm.at[b], bp_v)
        bpl = bp_v[...]
        s_b = bpl[0]
        s_e = bpl[1]
        a0 = s_b & (-8)
        nch = (s_e - a0 + CE2 - 1) >> 7

        @pl.loop(0, BN + 1)
        def _(d):
            acc2_v[pl.ds(d * 16, 16)] = z16
            den2_v[pl.ds(d * 16, 16)] = z16

        def chunk(ci, carry):
            lo = pl.multiple_of(a0 + ci * CE2, 8)
            c1 = pltpu.async_copy(srcs.at[pl.ds(lo, CE2)], sidx_v, s1)
            c2 = pltpu.async_copy(dsts.at[pl.ds(lo, CE2)], didx_v, s2)
            c1.wait()
            c2.wait()

            @pl.loop(0, CE2 // 16)
            def _(g):
                g16 = g * 16
                svec = sidx_v[pl.ds(g16, 16)]
                dvec = didx_v[pl.ds(g16, 16)]
                hs = plsc.load_gather(h2l_v, [svec])
                hd = plsc.load_gather(h2l_v, [dvec])
                sv = asv * hs + adv * hd
                al = jnp.maximum(sv, 0.2 * sv)
                ex = jnp.exp(al - c2v)
                pr = ex * hs
                dl = dvec - base
                jv = jnp.where((dl >= 0) & (dl < BN), dl, BN)
                for l in range(16):
                    dloc = jv[l]
                    plsc.addupdate(acc2_v.at[pl.ds(dloc * 16, 16)],
                                   jnp.full((16,), pr[l], _f32))
                    plsc.addupdate(den2_v.at[pl.ds(dloc * 16, 16)],
                                   jnp.full((16,), ex[l], _f32))
            return carry

        lax.fori_loop(0, nch, chunk, 0)
        pltpu.sync_copy(acc2_v.at[pl.ds(0, BN * 16)],
                        acc2h.at[pl.ds(base * 16, BN * 16)])
        pltpu.sync_copy(den2_v.at[pl.ds(0, BN * 16)],
                        den2h.at[pl.ds(base * 16, BN * 16)])
        return carry0

    lax.fori_loop(0, BPW, block, 0)


def kernel(x, edge_index, W1, att_src1, att_dst1, b1, W2, att_src2,
           att_dst2, b2):
    n = x.shape[0]
    e_tot = edge_index.shape[1] + n
    e2p = e_tot + 256
    phys = e2p + 128

    ei = edge_index.astype(_i32)
    loop = jnp.arange(n, dtype=_i32)
    src = jnp.concatenate([ei[0], loop])
    dst = jnp.concatenate([ei[1], loop])
    dst_s, src_s = lax.sort_key_val(dst, src)
    npad = phys - e_tot
    src_sp = jnp.concatenate([src_s, jnp.arange(npad, dtype=_i32) % n])
    dst_sp = jnp.concatenate([dst_s, jnp.full((npad,), NP - 1, _i32)])
    bounds = jnp.arange(NB + 1, dtype=_i32) * BN
    bp = jnp.minimum(jnp.searchsorted(dst_sp, bounds, side="left"),
                     e2p).astype(_i32)
    bpm = jnp.pad(jnp.stack([bp[:NB], bp[1:NB + 1]], axis=1),
                  ((0, 0), (0, 14)))

    x_p = jnp.pad(x, ((0, NP - n), (0, 0)))
    asf = att_src1.reshape(1, F)
    adf = att_dst1.reshape(1, F)
    ht, a1d, ms, md = _ka(x_p, W1, asf, adf)

    c8 = jnp.maximum(ms[0, :8] + md[0, :8], 0.0)
    scal1 = jnp.concatenate([c8, c8])
    acc1, den1f = _kb(ht, a1d.reshape(NP * 16), src_sp, dst_sp, bpm, scal1)
    den1 = den1f.reshape(NP, 16)

    b1r = b1.reshape(1, F)
    w2r = W2.reshape(1, F)
    att2 = jnp.concatenate(
        [att_src2.reshape(1, 1), att_dst2.reshape(1, 1),
         jnp.zeros((1, 14), _f32)], axis=1)
    h2, m2 = _kd(acc1, den1, b1r, w2r, att2)

    c2 = jnp.maximum(m2[0, 0] + m2[0, 1], 0.0)
    scal2 = jnp.concatenate(
        [jnp.stack([c2, att_src2[0, 0], att_dst2[0, 0]]),
         jnp.zeros((13,), _f32)])
    acc2f, den2f = _ke(h2.reshape(NP), src_sp, dst_sp, bpm, scal2)

    b2p = jnp.pad(b2.reshape(1, 1), ((0, 0), (0, 15)))
    out = _kf(acc2f.reshape(NP, 16), den2f.reshape(NP, 16), b2p)
    return out[:n]


# speedup vs baseline: 26.7791x; 2.0097x over previous
import dataclasses
import functools

import jax
import jax.numpy as jnp
from jax import lax
from jax.experimental import pallas as pl
from jax.experimental.pallas import tpu as pltpu
from jax.experimental.pallas import tpu_sc as plsc

N_IN = 64
HEADS = 8
HID = 64
F = HEADS * HID
FT = 640

BN = 128
NB = 416
NP = NB * BN
NWORK = 32
BPW = NB // NWORK
CE = 64
CE2 = 128
BA = 256

_f32 = jnp.float32
_i32 = jnp.int32


def _ka_body(x_ref, w_ref, asf_ref, adf_ref, ht_ref, ad_ref,
             ms_ref, md_ref):
    h = jnp.dot(x_ref[...], w_ref[...], preferred_element_type=_f32)
    ht_ref[:, :F] = h
    ts = h * asf_ref[...]
    td = h * adf_ref[...]
    zs = jnp.zeros((BA, 8), _f32)
    asb = jnp.concatenate(
        [jnp.sum(ts[:, i * HID:(i + 1) * HID], axis=1, keepdims=True)
         for i in range(HEADS)] + [zs], axis=1)
    adb = jnp.concatenate(
        [jnp.sum(td[:, i * HID:(i + 1) * HID], axis=1, keepdims=True)
         for i in range(HEADS)] + [zs], axis=1)
    ht_ref[:, F:F + 16] = asb
    ht_ref[:, F + 16:] = jnp.zeros((BA, FT - F - 16), _f32)
    ad_ref[...] = adb
    msb = jnp.max(asb, axis=0, keepdims=True)
    mdb = jnp.max(adb, axis=0, keepdims=True)
    i = pl.program_id(0)

    @pl.when(i == 0)
    def _():
        ms_ref[...] = msb
        md_ref[...] = mdb

    @pl.when(i != 0)
    def _():
        ms_ref[...] = jnp.maximum(ms_ref[...], msb)
        md_ref[...] = jnp.maximum(md_ref[...], mdb)


def _ka(x_p, w1, asf, adf):
    return pl.pallas_call(
        _ka_body,
        grid=(NP // BA,),
        in_specs=[
            pl.BlockSpec((BA, N_IN), lambda i: (i, 0)),
            pl.BlockSpec((N_IN, F), lambda i: (0, 0)),
            pl.BlockSpec((1, F), lambda i: (0, 0)),
            pl.BlockSpec((1, F), lambda i: (0, 0)),
        ],
        out_specs=[
            pl.BlockSpec((BA, FT), lambda i: (i, 0)),
            pl.BlockSpec((BA, 16), lambda i: (i, 0)),
            pl.BlockSpec((1, 16), lambda i: (0, 0)),
            pl.BlockSpec((1, 16), lambda i: (0, 0)),
        ],
        out_shape=[
            jax.ShapeDtypeStruct((NP, FT), _f32),
            jax.ShapeDtypeStruct((NP, 16), _f32),
            jax.ShapeDtypeStruct((1, 16), _f32),
            jax.ShapeDtypeStruct((1, 16), _f32),
        ],
    )(x_p, w1, asf, adf)


def _kd_body(acc_ref, den_ref, b1_ref, w2_ref, att_ref, h2_ref, m2_ref):
    den = den_ref[...]
    denb = jnp.concatenate(
        [jnp.broadcast_to(den[:, i:i + 1], (BA, HID)) for i in range(HEADS)],
        axis=1)
    y = acc_ref[...] / (denb + 1e-16) + b1_ref[...]
    y = jnp.where(y > 0, y, jnp.exp(jnp.minimum(y, 0.0)) - 1.0)
    h2 = jnp.sum(y * w2_ref[...], axis=1, keepdims=True)
    h2_ref[...] = h2
    s = att_ref[0, 0] * h2
    d = att_ref[0, 1] * h2
    m2b = jnp.concatenate(
        [jnp.max(s, axis=0, keepdims=True),
         jnp.max(d, axis=0, keepdims=True),
         jnp.zeros((1, 14), _f32)], axis=1)
    i = pl.program_id(0)

    @pl.when(i == 0)
    def _():
        m2_ref[...] = m2b

    @pl.when(i != 0)
    def _():
        m2_ref[...] = jnp.maximum(m2_ref[...], m2b)


def _kd(acc1, den1, b1r, w2r, att2):
    return pl.pallas_call(
        _kd_body,
        grid=(NP // BA,),
        in_specs=[
            pl.BlockSpec((BA, F), lambda i: (i, 0)),
            pl.BlockSpec((BA, 16), lambda i: (i, 0)),
            pl.BlockSpec((1, F), lambda i: (0, 0)),
            pl.BlockSpec((1, F), lambda i: (0, 0)),
            pl.BlockSpec((1, 16), lambda i: (0, 0)),
        ],
        out_specs=[
            pl.BlockSpec((BA, 1), lambda i: (i, 0)),
            pl.BlockSpec((1, 16), lambda i: (0, 0)),
        ],
        out_shape=[
            jax.ShapeDtypeStruct((NP, 1), _f32),
            jax.ShapeDtypeStruct((1, 16), _f32),
        ],
    )(acc1, den1, b1r, w2r, att2)


def _kf_body(acc_ref, den_ref, b2_ref, o_ref):
    o_ref[...] = (acc_ref[:, :1] / (den_ref[:, :1] + 1e-16)
                  + b2_ref[0, 0])


def _kf(acc2, den2, b2p):
    return pl.pallas_call(
        _kf_body,
        grid=(NP // BA,),
        in_specs=[
            pl.BlockSpec((BA, 16), lambda i: (i, 0)),
            pl.BlockSpec((BA, 16), lambda i: (i, 0)),
            pl.BlockSpec((1, 16), lambda i: (0, 0)),
        ],
        out_specs=pl.BlockSpec((BA, 1), lambda i: (i, 0)),
        out_shape=jax.ShapeDtypeStruct((NP, 1), _f32),
    )(acc2, den2, b2p)


_mesh = plsc.VectorSubcoreMesh(core_axis_name="c", subcore_axis_name="s")

_sc_params = pltpu.CompilerParams()
if "needs_layout_passes" in pltpu.CompilerParams.__dataclass_fields__:
    _sc_params = dataclasses.replace(_sc_params, needs_layout_passes=False)


@functools.partial(
    pl.kernel,
    out_type=[
        jax.ShapeDtypeStruct((NP, F), _f32),
        jax.ShapeDtypeStruct((NP * 16,), _f32),
    ],
    mesh=_mesh,
    scratch_types=[
        pltpu.VMEM((BN + 1, F), _f32),
        pltpu.VMEM(((BN + 1) * 16,), _f32),
        pltpu.VMEM((CE, FT), _f32),
        pltpu.VMEM((CE,), _i32),
        pltpu.VMEM((CE,), _i32),
        pltpu.VMEM(((BN + 1) * 16,), _f32),
        pltpu.VMEM((16,), _i32),
        pltpu.VMEM((16,), _f32),
        pltpu.VMEM((128,), _f32),
        pltpu.SemaphoreType.DMA,
        pltpu.SemaphoreType.DMA,
        pltpu.SemaphoreType.DMA,
    ],
    compiler_params=_sc_params,
)
def _kb(ht, a1d, srcs, dsts, bpm, scalh, acc1, den1,
        acc_v, den_v, rows_v, sidx_v, didx_v, adb_v,
        bp_v, scal_v, exb_v, s1, s2, s3):
    w = lax.axis_index("c") * 16 + lax.axis_index("s")
    pltpu.sync_copy(scalh, scal_v)
    z16 = jnp.zeros((16,), _f32)
    cvec = scal_v[...]

    def zero_acc():
        @pl.loop(0, BN + 1)
        def _(d):
            den_v[pl.ds(d * 16, 16)] = z16
            for i in range(F // 16):
                acc_v[d, pl.ds(i * 16, 16)] = z16

    zero_acc()

    def block(t, carry0):
        b = w + NWORK * t
        base = b * BN
        pltpu.sync_copy(bpm.at[b], bp_v)
        pltpu.sync_copy(a1d.at[pl.ds(base * 16, BN * 16)],
                        adb_v.at[pl.ds(0, BN * 16)])
        adb_v[pl.ds(BN * 16, 16)] = z16
        bpl = bp_v[...]
        s_b = bpl[0]
        s_e = bpl[1]
        a0 = s_b & (-8)
        nch = (s_e - a0 + CE - 1) >> 6

        def chunk(ci, carry):
            lo = pl.multiple_of(a0 + ci * CE, 8)
            c1 = pltpu.async_copy(srcs.at[pl.ds(lo, CE)], sidx_v, s1)
            c2 = pltpu.async_copy(dsts.at[pl.ds(lo, CE)], didx_v, s2)
            c1.wait()
            c3 = pltpu.async_copy(ht.at[sidx_v], rows_v, s3)
            c2.wait()
            c3.wait()

            @pl.loop(0, CE // 16)
            def _(g):
                g16 = g * 16
                dvec = didx_v[pl.ds(g16, 16)]
                dl = dvec - base
                jv = jnp.where((dl >= 0) & (dl < BN), dl, BN)
                for l in range(16):
                    j = g16 + l
                    dloc = jv[l]
                    sv = (rows_v[j, pl.ds(F, 16)]
                          + adb_v[pl.ds(dloc * 16, 16)])
                    al = jnp.maximum(sv, 0.2 * sv)
                    exrow = jnp.exp(al - cvec)
                    plsc.addupdate(den_v.at[pl.ds(dloc * 16, 16)], exrow)
                    for h in range(HEADS):
                        exb_v[pl.ds(h * 16, 16)] = jnp.full(
                            (16,), exrow[h], _f32)

                    @plsc.parallel_loop(0, F // 16, unroll=8)
                    def _(i):
                        off = i * 16
                        plsc.addupdate(
                            acc_v.at[dloc, pl.ds(off, 16)],
                            exb_v[pl.ds((i >> 2) * 16, 16)]
                            * rows_v[j, pl.ds(off, 16)])
            return carry

        lax.fori_loop(0, nch, chunk, 0)
        pltpu.sync_copy(acc_v.at[pl.ds(0, BN)], acc1.at[pl.ds(base, BN)])
        pltpu.sync_copy(den_v.at[pl.ds(0, BN * 16)],
                        den1.at[pl.ds(base * 16, BN * 16)])
        zero_acc()
        return carry0

    lax.fori_loop(0, BPW, block, 0)


@functools.partial(
    pl.kernel,
    out_type=[
        jax.ShapeDtypeStruct((NP * 16,), _f32),
        jax.ShapeDtypeStruct((NP * 16,), _f32),
    ],
    mesh=_mesh,
    scratch_types=[
        pltpu.VMEM((NP,), _f32),
        pltpu.VMEM(((BN + 1) * 16,), _f32),
        pltpu.VMEM(((BN + 1) * 16,), _f32),
        pltpu.VMEM((CE2,), _i32),
        pltpu.VMEM((CE2,), _i32),
        pltpu.VMEM((16,), _i32),
        pltpu.VMEM((16,), _f32),
        pltpu.SemaphoreType.DMA,
        pltpu.SemaphoreType.DMA,
    ],
    compiler_params=_sc_params,
)
def _ke(h2h, srcs, dsts, bpm, scalh, acc2h, den2h,
        h2l_v, acc2_v, den2_v, sidx_v, didx_v,
        bp_v, scal_v, s1, s2):
    w = lax.axis_index("c") * 16 + lax.axis_index("s")
    pltpu.sync_copy(scalh, scal_v)
    pltpu.sync_copy(h2h, h2l_v)
    z16 = jnp.zeros((16,), _f32)
    scl = scal_v[...]
    c2v = jnp.full((16,), scl[0], _f32)
    asv = jnp.full((16,), scl[1], _f32)
    adv = jnp.full((16,), scl[2], _f32)

    def block(t, carry0):
        b = w + NWORK * t
        base = b * BN
        pltpu.sync_copy(bpm.at[b], bp_v)
        bpl = bp_v[...]
        s_b = bpl[0]
        s_e = bpl[1]
        a0 = s_b & (-8)
        nch = (s_e - a0 + CE2 - 1) >> 7

        @pl.loop(0, BN + 1)
        def _(d):
            acc2_v[pl.ds(d * 16, 16)] = z16
            den2_v[pl.ds(d * 16, 16)] = z16

        def chunk(ci, carry):
            lo = pl.multiple_of(a0 + ci * CE2, 8)
            c1 = pltpu.async_copy(srcs.at[pl.ds(lo, CE2)], sidx_v, s1)
            c2 = pltpu.async_copy(dsts.at[pl.ds(lo, CE2)], didx_v, s2)
            c1.wait()
            c2.wait()

            @pl.loop(0, CE2 // 16)
            def _(g):
                g16 = g * 16
                svec = sidx_v[pl.ds(g16, 16)]
                dvec = didx_v[pl.ds(g16, 16)]
                hs = plsc.load_gather(h2l_v, [svec])
                hd = plsc.load_gather(h2l_v, [dvec])
                sv = asv * hs + adv * hd
                al = jnp.maximum(sv, 0.2 * sv)
                ex = jnp.exp(al - c2v)
                pr = ex * hs
                dl = dvec - base
                jv = jnp.where((dl >= 0) & (dl < BN), dl, BN)
                for l in range(16):
                    dloc = jv[l]
                    plsc.addupdate(acc2_v.at[pl.ds(dloc * 16, 16)],
                                   jnp.full((16,), pr[l], _f32))
                    plsc.addupdate(den2_v.at[pl.ds(dloc * 16, 16)],
                                   jnp.full((16,), ex[l], _f32))
            return carry

        lax.fori_loop(0, nch, chunk, 0)
        pltpu.sync_copy(acc2_v.at[pl.ds(0, BN * 16)],
                        acc2h.at[pl.ds(base * 16, BN * 16)])
        pltpu.sync_copy(den2_v.at[pl.ds(0, BN * 16)],
                        den2h.at[pl.ds(base * 16, BN * 16)])
        return carry0

    lax.fori_loop(0, BPW, block, 0)


def kernel(x, edge_index, W1, att_src1, att_dst1, b1, W2, att_src2,
           att_dst2, b2):
    n = x.shape[0]
    e_tot = edge_index.shape[1] + n
    e2p = e_tot + 256
    phys = e2p + 128

    ei = edge_index.astype(_i32)
    loop = jnp.arange(n, dtype=_i32)
    src = jnp.concatenate([ei[0], loop])
    dst = jnp.concatenate([ei[1], loop])
    dst_s, src_s = lax.sort_key_val(dst, src)
    npad = phys - e_tot
    src_sp = jnp.concatenate([src_s, jnp.arange(npad, dtype=_i32) % n])
    dst_sp = jnp.concatenate([dst_s, jnp.full((npad,), NP - 1, _i32)])
    bounds = jnp.arange(NB + 1, dtype=_i32) * BN
    bp = jnp.minimum(jnp.searchsorted(dst_sp, bounds, side="left"),
                     e2p).astype(_i32)
    bpm = jnp.pad(jnp.stack([bp[:NB], bp[1:NB + 1]], axis=1),
                  ((0, 0), (0, 14)))

    x_p = jnp.pad(x, ((0, NP - n), (0, 0)))
    asf = att_src1.reshape(1, F)
    adf = att_dst1.reshape(1, F)
    ht, a1d, ms, md = _ka(x_p, W1, asf, adf)

    c8 = jnp.maximum(ms[0, :8] + md[0, :8], 0.0)
    scal1 = jnp.concatenate([c8, c8])
    acc1, den1f = _kb(ht, a1d.reshape(NP * 16), src_sp, dst_sp, bpm, scal1)
    den1 = den1f.reshape(NP, 16)

    b1r = b1.reshape(1, F)
    w2r = W2.reshape(1, F)
    att2 = jnp.concatenate(
        [att_src2.reshape(1, 1), att_dst2.reshape(1, 1),
         jnp.zeros((1, 14), _f32)], axis=1)
    h2, m2 = _kd(acc1, den1, b1r, w2r, att2)

    c2 = jnp.maximum(m2[0, 0] + m2[0, 1], 0.0)
    scal2 = jnp.concatenate(
        [jnp.stack([c2, att_src2[0, 0], att_dst2[0, 0]]),
         jnp.zeros((13,), _f32)])
    acc2f, den2f = _ke(h2.reshape(NP), src_sp, dst_sp, bpm, scal2)

    b2p = jnp.pad(b2.reshape(1, 1), ((0, 0), (0, 15)))
    out = _kf(acc2f.reshape(NP, 16), den2f.reshape(NP, 16), b2p)
    return out[:n]
